# Initial kernel scaffold; baseline (speedup 1.0000x reference)
import jax
import jax.numpy as jnp
from jax.experimental import pallas as pl


def kernel(x, signs, centroids):
    raise NotImplementedError("write your pallas kernel here")


# speedup vs baseline: 17.4184x; 17.4184x over previous
import numpy as np
import jax
import jax.numpy as jnp
from jax.experimental import pallas as pl
from jax.experimental.pallas import tpu as pltpu

D = 128
K = 16
BLOCK_ROWS = 1024


def _hadamard(d: int) -> np.ndarray:
    h = np.array([[1.0]], dtype=np.float32)
    while h.shape[0] < d:
        h = np.block([[h, h], [h, -h]])
    return h.astype(np.float32)


_T = _hadamard(D)


def _body(mids_ref, dc_ref, c0_ref, x_ref, a_ref, b_ref, xhat_ref, idx_ref):
    x = x_ref[...]
    y = jax.lax.dot(
        x, a_ref[...],
        precision=jax.lax.Precision.HIGHEST,
        preferred_element_type=jnp.float32,
    )
    idx = jnp.zeros(y.shape, jnp.int32)
    yhat = jnp.full(y.shape, c0_ref[0], jnp.float32)
    for k in range(K - 1):
        gt = y > mids_ref[k]
        idx = idx + gt.astype(jnp.int32)
        yhat = yhat + jnp.where(gt, dc_ref[k], 0.0)
    xhat = jax.lax.dot(
        yhat, b_ref[...],
        precision=jax.lax.Precision.HIGHEST,
        preferred_element_type=jnp.float32,
    )
    xhat_ref[...] = xhat
    idx_ref[...] = idx


def kernel(x, signs, centroids):
    n, d = x.shape
    scale = 1.0 / jnp.sqrt(jnp.asarray(d, jnp.float32))
    t = jnp.asarray(_T)
    a = signs[:, None] * t * scale
    b = t * (scale * signs[None, :])
    mids = 0.5 * (centroids[1:] + centroids[:-1])
    dc = centroids[1:] - centroids[:-1]
    c0 = centroids[:1]

    grid = (n // BLOCK_ROWS,)
    xhat, idx = pl.pallas_call(
        _body,
        grid=grid,
        in_specs=[
            pl.BlockSpec(memory_space=pltpu.SMEM),
            pl.BlockSpec(memory_space=pltpu.SMEM),
            pl.BlockSpec(memory_space=pltpu.SMEM),
            pl.BlockSpec((BLOCK_ROWS, d), lambda i: (i, 0)),
            pl.BlockSpec((d, d), lambda i: (0, 0)),
            pl.BlockSpec((d, d), lambda i: (0, 0)),
        ],
        out_specs=[
            pl.BlockSpec((BLOCK_ROWS, d), lambda i: (i, 0)),
            pl.BlockSpec((BLOCK_ROWS, d), lambda i: (i, 0)),
        ],
        out_shape=[
            jax.ShapeDtypeStruct((n, d), jnp.float32),
            jax.ShapeDtypeStruct((n, d), jnp.int32),
        ],
        compiler_params=pltpu.CompilerParams(
            dimension_semantics=("arbitrary",),
        ),
    )(mids, dc, c0, x, a, b)
    return xhat, idx


# speedup vs baseline: 20.3468x; 1.1681x over previous
import numpy as np
import jax
import jax.numpy as jnp
from jax.experimental import pallas as pl
from jax.experimental.pallas import tpu as pltpu

D = 128
K = 16
BLOCK_ROWS = 1024


def _hadamard(d: int) -> np.ndarray:
    h = np.array([[1.0]], dtype=np.float32)
    while h.shape[0] < d:
        h = np.block([[h, h], [h, -h]])
    return h.astype(np.float32)


_T = _hadamard(D)


def _body(mids_ref, cents_ref, x_ref, a_ref, b_ref, xhat_ref, idx_ref):
    x = x_ref[...]
    y = jax.lax.dot(
        x, a_ref[...],
        precision=jax.lax.Precision.HIGHEST,
        preferred_element_type=jnp.float32,
    )
    mid = [mids_ref[k] for k in range(K - 1)]

    m3 = y > mid[7]
    b2 = jnp.where(m3, mid[11], mid[3])
    m2 = y > b2
    b1 = jnp.where(m3,
                   jnp.where(m2, mid[13], mid[9]),
                   jnp.where(m2, mid[5], mid[1]))
    m1 = y > b1
    b0 = jnp.where(m3,
                   jnp.where(m2,
                             jnp.where(m1, mid[14], mid[12]),
                             jnp.where(m1, mid[10], mid[8])),
                   jnp.where(m2,
                             jnp.where(m1, mid[6], mid[4]),
                             jnp.where(m1, mid[2], mid[0])))
    m0 = y > b0
    idx = (m3.astype(jnp.int32) * 8 + m2.astype(jnp.int32) * 4
           + m1.astype(jnp.int32) * 2 + m0.astype(jnp.int32))
    c = [cents_ref[k] for k in range(K)]
    yhat = jnp.where(
        m3,
        jnp.where(m2,
                  jnp.where(m1,
                            jnp.where(m0, c[15], c[14]),
                            jnp.where(m0, c[13], c[12])),
                  jnp.where(m1,
                            jnp.where(m0, c[11], c[10]),
                            jnp.where(m0, c[9], c[8]))),
        jnp.where(m2,
                  jnp.where(m1,
                            jnp.where(m0, c[7], c[6]),
                            jnp.where(m0, c[5], c[4])),
                  jnp.where(m1,
                            jnp.where(m0, c[3], c[2]),
                            jnp.where(m0, c[1], c[0]))))
    xhat = jax.lax.dot(
        yhat, b_ref[...],
        precision=jax.lax.Precision.HIGHEST,
        preferred_element_type=jnp.float32,
    )
    xhat_ref[...] = xhat
    idx_ref[...] = idx


def kernel(x, signs, centroids):
    n, d = x.shape
    scale = 1.0 / jnp.sqrt(jnp.asarray(d, jnp.float32))
    t = jnp.asarray(_T)
    a = signs[:, None] * t * scale
    b = t * (scale * signs[None, :])
    mids = 0.5 * (centroids[1:] + centroids[:-1])

    grid = (n // BLOCK_ROWS,)
    xhat, idx = pl.pallas_call(
        _body,
        grid=grid,
        in_specs=[
            pl.BlockSpec(memory_space=pltpu.SMEM),
            pl.BlockSpec(memory_space=pltpu.SMEM),
            pl.BlockSpec((BLOCK_ROWS, d), lambda i: (i, 0)),
            pl.BlockSpec((d, d), lambda i: (0, 0)),
            pl.BlockSpec((d, d), lambda i: (0, 0)),
        ],
        out_specs=[
            pl.BlockSpec((BLOCK_ROWS, d), lambda i: (i, 0)),
            pl.BlockSpec((BLOCK_ROWS, d), lambda i: (i, 0)),
        ],
        out_shape=[
            jax.ShapeDtypeStruct((n, d), jnp.float32),
            jax.ShapeDtypeStruct((n, d), jnp.int32),
        ],
        compiler_params=pltpu.CompilerParams(
            dimension_semantics=("arbitrary",),
        ),
    )(mids, centroids, x, a, b)
    return xhat, idx


# speedup vs baseline: 35.5651x; 1.7479x over previous
import numpy as np
import jax
import jax.numpy as jnp
from jax.experimental import pallas as pl
from jax.experimental.pallas import tpu as pltpu

D = 128
K = 16
BLOCK_ROWS = 1024


def _hadamard(d: int) -> np.ndarray:
    h = np.array([[1.0]], dtype=np.float32)
    while h.shape[0] < d:
        h = np.block([[h, h], [h, -h]])
    return h.astype(np.float32)


_T = _hadamard(D)


def _body(mids_ref, cents_ref, x_ref, a_ref, b_ref, xhat_ref, idx_ref):
    x = x_ref[...]
    y = jax.lax.dot(
        x, a_ref[...],
        precision=jax.lax.Precision.HIGHEST,
        preferred_element_type=jnp.float32,
    )
    mid = [mids_ref[k] for k in range(K - 1)]

    m3 = y > mid[7]
    b2 = jnp.where(m3, mid[11], mid[3])
    m2 = y > b2
    b1 = jnp.where(m3,
                   jnp.where(m2, mid[13], mid[9]),
                   jnp.where(m2, mid[5], mid[1]))
    m1 = y > b1
    b0 = jnp.where(m3,
                   jnp.where(m2,
                             jnp.where(m1, mid[14], mid[12]),
                             jnp.where(m1, mid[10], mid[8])),
                   jnp.where(m2,
                             jnp.where(m1, mid[6], mid[4]),
                             jnp.where(m1, mid[2], mid[0])))
    m0 = y > b0
    zero = jnp.zeros(y.shape, jnp.int32)
    idx = (jnp.where(m3, 8, zero) + jnp.where(m2, 4, zero)
           + jnp.where(m1, 2, zero) + jnp.where(m0, 1, zero))
    c = [cents_ref[k] for k in range(K)]
    clo = jnp.where(m3,
                    jnp.where(m2,
                              jnp.where(m1, c[14], c[12]),
                              jnp.where(m1, c[10], c[8])),
                    jnp.where(m2,
                              jnp.where(m1, c[6], c[4]),
                              jnp.where(m1, c[2], c[0])))
    yhat = jnp.where(m0, 2.0 * b0 - clo, clo)
    xhat = jax.lax.dot(
        yhat.astype(jnp.bfloat16), b_ref[...],
        preferred_element_type=jnp.float32,
    )
    xhat_ref[...] = xhat
    idx_ref[...] = idx


def kernel(x, signs, centroids):
    n, d = x.shape
    scale = 1.0 / jnp.sqrt(jnp.asarray(d, jnp.float32))
    t = jnp.asarray(_T)
    a = signs[:, None] * t * scale
    b = (t * (scale * signs[None, :])).astype(jnp.bfloat16)
    mids = 0.5 * (centroids[1:] + centroids[:-1])

    grid = (n // BLOCK_ROWS,)
    xhat, idx = pl.pallas_call(
        _body,
        grid=grid,
        in_specs=[
            pl.BlockSpec(memory_space=pltpu.SMEM),
            pl.BlockSpec(memory_space=pltpu.SMEM),
            pl.BlockSpec((BLOCK_ROWS, d), lambda i: (i, 0)),
            pl.BlockSpec((d, d), lambda i: (0, 0)),
            pl.BlockSpec((d, d), lambda i: (0, 0)),
        ],
        out_specs=[
            pl.BlockSpec((BLOCK_ROWS, d), lambda i: (i, 0)),
            pl.BlockSpec((BLOCK_ROWS, d), lambda i: (i, 0)),
        ],
        out_shape=[
            jax.ShapeDtypeStruct((n, d), jnp.float32),
            jax.ShapeDtypeStruct((n, d), jnp.int32),
        ],
        compiler_params=pltpu.CompilerParams(
            dimension_semantics=("arbitrary",),
        ),
    )(mids, centroids, x, a, b)
    return xhat, idx


# speedup vs baseline: 39.1933x; 1.1020x over previous
import numpy as np
import jax
import jax.numpy as jnp
from jax.experimental import pallas as pl
from jax.experimental.pallas import tpu as pltpu

D = 128
K = 16
BLOCK_ROWS = 1024


def _hadamard(d: int) -> np.ndarray:
    h = np.array([[1.0]], dtype=np.float32)
    while h.shape[0] < d:
        h = np.block([[h, h], [h, -h]])
    return h.astype(np.float32)


_T = _hadamard(D)


_TWO_SCALE = float(2.0 / np.sqrt(np.float32(D)))


def _body(mids_ref, cents_ref, x_ref, a_ref, b_ref, xhat_ref, idx_ref):
    x = x_ref[...]
    xh = x.astype(jnp.bfloat16)
    xl = (x - xh.astype(jnp.float32)).astype(jnp.bfloat16)
    a = a_ref[...]
    y = (jax.lax.dot(xh, a, preferred_element_type=jnp.float32)
         + jax.lax.dot(xl, a, preferred_element_type=jnp.float32))
    mid = [mids_ref[k] for k in range(K - 1)]

    m3 = y > mid[7]
    b2 = jnp.where(m3, mid[11], mid[3])
    m2 = y > b2
    b1 = jnp.where(m3,
                   jnp.where(m2, mid[13], mid[9]),
                   jnp.where(m2, mid[5], mid[1]))
    m1 = y > b1
    b0 = jnp.where(m3,
                   jnp.where(m2,
                             jnp.where(m1, mid[14], mid[12]),
                             jnp.where(m1, mid[10], mid[8])),
                   jnp.where(m2,
                             jnp.where(m1, mid[6], mid[4]),
                             jnp.where(m1, mid[2], mid[0])))
    m0 = y > b0
    zero = jnp.zeros(y.shape, jnp.int32)
    idx = (jnp.where(m3, 8, zero) + jnp.where(m2, 4, zero)
           + jnp.where(m1, 2, zero) + jnp.where(m0, 1, zero))
    c = [cents_ref[k] for k in range(K)]
    clo = jnp.where(m3,
                    jnp.where(m2,
                              jnp.where(m1, c[14], c[12]),
                              jnp.where(m1, c[10], c[8])),
                    jnp.where(m2,
                              jnp.where(m1, c[6], c[4]),
                              jnp.where(m1, c[2], c[0])))
    yhat = jnp.where(m0, _TWO_SCALE * b0 - clo, clo)
    xhat = jax.lax.dot(
        yhat.astype(jnp.bfloat16), b_ref[...],
        preferred_element_type=jnp.float32,
    )
    xhat_ref[...] = xhat
    idx_ref[...] = idx


def kernel(x, signs, centroids):
    n, d = x.shape
    scale = 1.0 / jnp.sqrt(jnp.asarray(d, jnp.float32))
    t = jnp.asarray(_T)
    a = (signs[:, None] * t).astype(jnp.bfloat16)
    b = (t * (scale * signs[None, :])).astype(jnp.bfloat16)
    mids = 0.5 * (centroids[1:] + centroids[:-1]) / scale

    grid = (n // BLOCK_ROWS,)
    xhat, idx = pl.pallas_call(
        _body,
        grid=grid,
        in_specs=[
            pl.BlockSpec(memory_space=pltpu.SMEM),
            pl.BlockSpec(memory_space=pltpu.SMEM),
            pl.BlockSpec((BLOCK_ROWS, d), lambda i: (i, 0)),
            pl.BlockSpec((d, d), lambda i: (0, 0)),
            pl.BlockSpec((d, d), lambda i: (0, 0)),
        ],
        out_specs=[
            pl.BlockSpec((BLOCK_ROWS, d), lambda i: (i, 0)),
            pl.BlockSpec((BLOCK_ROWS, d), lambda i: (i, 0)),
        ],
        out_shape=[
            jax.ShapeDtypeStruct((n, d), jnp.float32),
            jax.ShapeDtypeStruct((n, d), jnp.int32),
        ],
        compiler_params=pltpu.CompilerParams(
            dimension_semantics=("arbitrary",),
        ),
    )(mids, centroids, x, a, b)
    return xhat, idx


# speedup vs baseline: 51.7145x; 1.3195x over previous
import numpy as np
import jax
import jax.numpy as jnp
from jax.experimental import pallas as pl
from jax.experimental.pallas import tpu as pltpu

D = 128
K = 16
BLOCK_ROWS = 2048


def _hadamard(d: int) -> np.ndarray:
    h = np.array([[1.0]], dtype=np.float32)
    while h.shape[0] < d:
        h = np.block([[h, h], [h, -h]])
    return h.astype(np.float32)


_T = _hadamard(D)


_TWO_SCALE = float(2.0 / np.sqrt(np.float32(D)))


def _body(mids_ref, cents_ref, x_ref, a_ref, b_ref, xhat_ref, idx_ref):
    x = x_ref[...]
    xh = x.astype(jnp.bfloat16)
    xl = (x - xh.astype(jnp.float32)).astype(jnp.bfloat16)
    a = a_ref[...]
    y = (jax.lax.dot(xh, a, preferred_element_type=jnp.float32)
         + jax.lax.dot(xl, a, preferred_element_type=jnp.float32))
    mid = [mids_ref[k] for k in range(K - 1)]

    m3 = y > mid[7]
    b2 = jnp.where(m3, mid[11], mid[3])
    m2 = y > b2
    b1 = jnp.where(m3,
                   jnp.where(m2, mid[13], mid[9]),
                   jnp.where(m2, mid[5], mid[1]))
    m1 = y > b1
    b0 = jnp.where(m3,
                   jnp.where(m2,
                             jnp.where(m1, mid[14], mid[12]),
                             jnp.where(m1, mid[10], mid[8])),
                   jnp.where(m2,
                             jnp.where(m1, mid[6], mid[4]),
                             jnp.where(m1, mid[2], mid[0])))
    m0 = y > b0
    zero = jnp.zeros(y.shape, jnp.int32)
    idx = (jnp.where(m3, 8, zero) + jnp.where(m2, 4, zero)
           + jnp.where(m1, 2, zero) + jnp.where(m0, 1, zero))
    c = [cents_ref[k] for k in range(K)]
    clo = jnp.where(m3,
                    jnp.where(m2,
                              jnp.where(m1, c[14], c[12]),
                              jnp.where(m1, c[10], c[8])),
                    jnp.where(m2,
                              jnp.where(m1, c[6], c[4]),
                              jnp.where(m1, c[2], c[0])))
    yhat = jnp.where(m0, _TWO_SCALE * b0 - clo, clo)
    xhat = jax.lax.dot(
        yhat.astype(jnp.bfloat16), b_ref[...],
        preferred_element_type=jnp.float32,
    )
    xhat_ref[...] = xhat
    idx_ref[...] = idx


def kernel(x, signs, centroids):
    n, d = x.shape
    scale = 1.0 / jnp.sqrt(jnp.asarray(d, jnp.float32))
    t = jnp.asarray(_T)
    a = (signs[:, None] * t).astype(jnp.bfloat16)
    b = (t * (scale * signs[None, :])).astype(jnp.bfloat16)
    mids = 0.5 * (centroids[1:] + centroids[:-1]) / scale

    grid = (n // BLOCK_ROWS,)
    xhat, idx = pl.pallas_call(
        _body,
        grid=grid,
        in_specs=[
            pl.BlockSpec(memory_space=pltpu.SMEM),
            pl.BlockSpec(memory_space=pltpu.SMEM),
            pl.BlockSpec((BLOCK_ROWS, d), lambda i: (i, 0)),
            pl.BlockSpec((d, d), lambda i: (0, 0)),
            pl.BlockSpec((d, d), lambda i: (0, 0)),
        ],
        out_specs=[
            pl.BlockSpec((BLOCK_ROWS, d), lambda i: (i, 0)),
            pl.BlockSpec((BLOCK_ROWS, d), lambda i: (i, 0)),
        ],
        out_shape=[
            jax.ShapeDtypeStruct((n, d), jnp.float32),
            jax.ShapeDtypeStruct((n, d), jnp.int32),
        ],
        compiler_params=pltpu.CompilerParams(
            dimension_semantics=("arbitrary",),
        ),
    )(mids, centroids, x, a, b)
    return xhat, idx


# speedup vs baseline: 63.0956x; 1.2201x over previous
import numpy as np
import jax
import jax.numpy as jnp
from jax.experimental import pallas as pl
from jax.experimental.pallas import tpu as pltpu

D = 128
K = 16
BLOCK_ROWS = 4096


def _hadamard(d: int) -> np.ndarray:
    h = np.array([[1.0]], dtype=np.float32)
    while h.shape[0] < d:
        h = np.block([[h, h], [h, -h]])
    return h.astype(np.float32)


_T = _hadamard(D)


_TWO_SCALE = float(2.0 / np.sqrt(np.float32(D)))


def _body(mids_ref, cents_ref, x_ref, a_ref, b_ref, xhat_ref, idx_ref):
    x = x_ref[...]
    xh = x.astype(jnp.bfloat16)
    xl = (x - xh.astype(jnp.float32)).astype(jnp.bfloat16)
    a = a_ref[...]
    y = (jax.lax.dot(xh, a, preferred_element_type=jnp.float32)
         + jax.lax.dot(xl, a, preferred_element_type=jnp.float32))
    mid = [mids_ref[k] for k in range(K - 1)]

    m3 = y > mid[7]
    b2 = jnp.where(m3, mid[11], mid[3])
    m2 = y > b2
    b1 = jnp.where(m3,
                   jnp.where(m2, mid[13], mid[9]),
                   jnp.where(m2, mid[5], mid[1]))
    m1 = y > b1
    b0 = jnp.where(m3,
                   jnp.where(m2,
                             jnp.where(m1, mid[14], mid[12]),
                             jnp.where(m1, mid[10], mid[8])),
                   jnp.where(m2,
                             jnp.where(m1, mid[6], mid[4]),
                             jnp.where(m1, mid[2], mid[0])))
    m0 = y > b0
    zero = jnp.zeros(y.shape, jnp.int32)
    idx = (jnp.where(m3, 8, zero) + jnp.where(m2, 4, zero)
           + jnp.where(m1, 2, zero) + jnp.where(m0, 1, zero))
    c = [cents_ref[k] for k in range(K)]
    clo = jnp.where(m3,
                    jnp.where(m2,
                              jnp.where(m1, c[14], c[12]),
                              jnp.where(m1, c[10], c[8])),
                    jnp.where(m2,
                              jnp.where(m1, c[6], c[4]),
                              jnp.where(m1, c[2], c[0])))
    yhat = jnp.where(m0, _TWO_SCALE * b0 - clo, clo)
    xhat = jax.lax.dot(
        yhat.astype(jnp.bfloat16), b_ref[...],
        preferred_element_type=jnp.float32,
    )
    xhat_ref[...] = xhat
    idx_ref[...] = idx


def kernel(x, signs, centroids):
    n, d = x.shape
    scale = 1.0 / jnp.sqrt(jnp.asarray(d, jnp.float32))
    t = jnp.asarray(_T)
    a = (signs[:, None] * t).astype(jnp.bfloat16)
    b = (t * (scale * signs[None, :])).astype(jnp.bfloat16)
    mids = 0.5 * (centroids[1:] + centroids[:-1]) / scale

    grid = (n // BLOCK_ROWS,)
    xhat, idx = pl.pallas_call(
        _body,
        grid=grid,
        in_specs=[
            pl.BlockSpec(memory_space=pltpu.SMEM),
            pl.BlockSpec(memory_space=pltpu.SMEM),
            pl.BlockSpec((BLOCK_ROWS, d), lambda i: (i, 0)),
            pl.BlockSpec((d, d), lambda i: (0, 0)),
            pl.BlockSpec((d, d), lambda i: (0, 0)),
        ],
        out_specs=[
            pl.BlockSpec((BLOCK_ROWS, d), lambda i: (i, 0)),
            pl.BlockSpec((BLOCK_ROWS, d), lambda i: (i, 0)),
        ],
        out_shape=[
            jax.ShapeDtypeStruct((n, d), jnp.float32),
            jax.ShapeDtypeStruct((n, d), jnp.int32),
        ],
        compiler_params=pltpu.CompilerParams(
            dimension_semantics=("arbitrary",),
        ),
    )(mids, centroids, x, a, b)
    return xhat, idx


# speedup vs baseline: 64.9959x; 1.0301x over previous
import numpy as np
import jax
import jax.numpy as jnp
from jax.experimental import pallas as pl
from jax.experimental.pallas import tpu as pltpu

D = 128
K = 16
BLOCK_ROWS = 8192


def _hadamard(d: int) -> np.ndarray:
    h = np.array([[1.0]], dtype=np.float32)
    while h.shape[0] < d:
        h = np.block([[h, h], [h, -h]])
    return h.astype(np.float32)


_T = _hadamard(D)


_TWO_SCALE = float(2.0 / np.sqrt(np.float32(D)))


def _body(mids_ref, cents_ref, x_ref, a_ref, b_ref, xhat_ref, idx_ref):
    x = x_ref[...]
    xh = x.astype(jnp.bfloat16)
    xl = (x - xh.astype(jnp.float32)).astype(jnp.bfloat16)
    a = a_ref[...]
    y = (jax.lax.dot(xh, a, preferred_element_type=jnp.float32)
         + jax.lax.dot(xl, a, preferred_element_type=jnp.float32))
    mid = [mids_ref[k] for k in range(K - 1)]

    m3 = y > mid[7]
    b2 = jnp.where(m3, mid[11], mid[3])
    m2 = y > b2
    b1 = jnp.where(m3,
                   jnp.where(m2, mid[13], mid[9]),
                   jnp.where(m2, mid[5], mid[1]))
    m1 = y > b1
    b0 = jnp.where(m3,
                   jnp.where(m2,
                             jnp.where(m1, mid[14], mid[12]),
                             jnp.where(m1, mid[10], mid[8])),
                   jnp.where(m2,
                             jnp.where(m1, mid[6], mid[4]),
                             jnp.where(m1, mid[2], mid[0])))
    m0 = y > b0
    zero = jnp.zeros(y.shape, jnp.int32)
    idx = (jnp.where(m3, 8, zero) + jnp.where(m2, 4, zero)
           + jnp.where(m1, 2, zero) + jnp.where(m0, 1, zero))
    c = [cents_ref[k] for k in range(K)]
    clo = jnp.where(m3,
                    jnp.where(m2,
                              jnp.where(m1, c[14], c[12]),
                              jnp.where(m1, c[10], c[8])),
                    jnp.where(m2,
                              jnp.where(m1, c[6], c[4]),
                              jnp.where(m1, c[2], c[0])))
    yhat = jnp.where(m0, _TWO_SCALE * b0 - clo, clo)
    xhat = jax.lax.dot(
        yhat.astype(jnp.bfloat16), b_ref[...],
        preferred_element_type=jnp.float32,
    )
    xhat_ref[...] = xhat
    idx_ref[...] = idx


def kernel(x, signs, centroids):
    n, d = x.shape
    scale = 1.0 / jnp.sqrt(jnp.asarray(d, jnp.float32))
    t = jnp.asarray(_T)
    a = (signs[:, None] * t).astype(jnp.bfloat16)
    b = (t * (scale * signs[None, :])).astype(jnp.bfloat16)
    mids = 0.5 * (centroids[1:] + centroids[:-1]) / scale

    grid = (n // BLOCK_ROWS,)
    xhat, idx = pl.pallas_call(
        _body,
        grid=grid,
        in_specs=[
            pl.BlockSpec(memory_space=pltpu.SMEM),
            pl.BlockSpec(memory_space=pltpu.SMEM),
            pl.BlockSpec((BLOCK_ROWS, d), lambda i: (i, 0)),
            pl.BlockSpec((d, d), lambda i: (0, 0)),
            pl.BlockSpec((d, d), lambda i: (0, 0)),
        ],
        out_specs=[
            pl.BlockSpec((BLOCK_ROWS, d), lambda i: (i, 0)),
            pl.BlockSpec((BLOCK_ROWS, d), lambda i: (i, 0)),
        ],
        out_shape=[
            jax.ShapeDtypeStruct((n, d), jnp.float32),
            jax.ShapeDtypeStruct((n, d), jnp.int32),
        ],
        compiler_params=pltpu.CompilerParams(
            dimension_semantics=("arbitrary",),
        ),
    )(mids, centroids, x, a, b)
    return xhat, idx
